# baseline (speedup 1.0000x reference)
import functools

import jax
import jax.numpy as jnp
from jax import lax
from jax.experimental import pallas as pl
from jax.experimental.pallas import tpu as pltpu
from jax.experimental.pallas import tpu_sc as plsc

NUM_EMB = 1000000
D = 32
B = 16384

NC = 2
NS = 16
NW = NC * NS
BPW = B // NW
CH = 4
CW = BPW // CH

_mesh = plsc.VectorSubcoreMesh(core_axis_name="c", subcore_axis_name="s")


@functools.partial(
    pl.kernel,
    out_type=jax.ShapeDtypeStruct((B, D), jnp.float32),
    mesh=_mesh,
    scratch_types=[
        pltpu.VMEM((CH, CW), jnp.int32),
        pltpu.VMEM((BPW, D), jnp.float32),
        pltpu.SemaphoreType.DMA,
    ],
    compiler_params=pltpu.CompilerParams(use_tc_tiling_on_sc=False),
)
def _sc_gather(ids_hbm, table_hbm, out_hbm, idx_v, rows_v, sem):
    wid = lax.axis_index("s") * NC + lax.axis_index("c")
    pltpu.sync_copy(ids_hbm.at[wid], idx_v)
    copies = []
    for j in range(CH):
        copies.append(
            pltpu.async_copy(
                table_hbm.at[idx_v.at[j]],
                rows_v.at[pl.ds(j * CW, CW)],
                sem,
            )
        )
    for c in copies:
        c.wait()
    pltpu.sync_copy(rows_v, out_hbm.at[pl.ds(wid * BPW, BPW)])


def _mm_body(x_ref, w_ref, b_ref, o_ref):
    o_ref[...] = (
        lax.dot_general(
            x_ref[...], w_ref[...], (((1,), (1,)), ((), ())),
            preferred_element_type=jnp.float32,
        )
        + b_ref[...]
    )


_GR = 8
_mm = pl.pallas_call(
    _mm_body,
    out_shape=jax.ShapeDtypeStruct((B, D), jnp.float32),
    grid=(_GR,),
    in_specs=[
        pl.BlockSpec((B // _GR, D), lambda i: (i, 0)),
        pl.BlockSpec((D, D), lambda i: (0, 0)),
        pl.BlockSpec((1, D), lambda i: (0, 0)),
    ],
    out_specs=pl.BlockSpec((B // _GR, D), lambda i: (i, 0)),
)


def kernel(appearance_ids, table, W, b):
    ids = appearance_ids.astype(jnp.int32).reshape(NW, CH, CW)
    emb = _sc_gather(ids, table)
    return _mm(emb, W, b.reshape(1, D))


# speedup vs baseline: 3.5091x; 3.5091x over previous
import functools

import jax
import jax.numpy as jnp
from jax import lax
from jax.experimental import pallas as pl
from jax.experimental.pallas import tpu as pltpu
from jax.experimental.pallas import tpu_sc as plsc

NUM_EMB = 1000000
D = 32
B = 16384
DP = 128

NC = 2
NS = 16
NW = NC * NS
BPW = B // NW
G = 16
NG = BPW // G

_mesh = plsc.VectorSubcoreMesh(core_axis_name="c", subcore_axis_name="s")


@functools.partial(
    pl.kernel,
    out_type=jax.ShapeDtypeStruct((B, DP), jnp.float32),
    mesh=_mesh,
    scratch_types=[
        pltpu.VMEM((BPW,), jnp.int32),
        pltpu.VMEM((G, D, DP), jnp.float32),
        pltpu.VMEM((G, DP), jnp.float32),
        pltpu.SemaphoreType.DMA,
        pltpu.SemaphoreType.DMA,
    ],
    compiler_params=pltpu.CompilerParams(
        use_tc_tiling_on_sc=True, needs_layout_passes=False
    ),
)
def _sc_gather(ids_hbm, tabT_hbm, out_hbm, ids_v, sbuf, slab, semi, sems):
    wid = lax.axis_index("s") * NC + lax.axis_index("c")
    base = wid * BPW
    pltpu.async_copy(ids_hbm.at[pl.ds(base, BPW)], ids_v, semi).wait()

    lanes = lax.iota(jnp.int32, 16)

    def group(g, _):
        idv = ids_v[pl.ds(g * G, G)]
        cps = []
        for e in range(G):
            sid = idv[e]
            off = pl.multiple_of((sid >> 7) * 128, 128)
            cps.append(
                pltpu.async_copy(
                    tabT_hbm.at[:, pl.ds(off, 128)], sbuf.at[e], sems
                )
            )
        for cp in cps:
            cp.wait()
        r_vec = lax.bitwise_and(idv, jnp.int32(127))
        for j in range(D):
            j_vec = jnp.full((16,), j, jnp.int32)
            vals = plsc.load_gather(sbuf, [lanes, j_vec, r_vec])
            plsc.store_scatter(slab, [lanes, j_vec], vals)
        pltpu.sync_copy(slab, out_hbm.at[pl.ds(base + g * G, G), :])
        return 0

    lax.fori_loop(0, NG, group, 0, unroll=False)


def _mm_body(x_ref, w_ref, b_ref, o_ref):
    xs = x_ref[...][:, 0:D]
    o_ref[...] = (
        lax.dot_general(
            w_ref[...], xs, (((1,), (1,)), ((), ())),
            preferred_element_type=jnp.float32,
        )
        + b_ref[...]
    )


_GR = 8
_mm = pl.pallas_call(
    _mm_body,
    out_shape=jax.ShapeDtypeStruct((D, B), jnp.float32),
    grid=(_GR,),
    in_specs=[
        pl.BlockSpec((B // _GR, DP), lambda i: (i, 0)),
        pl.BlockSpec((D, D), lambda i: (0, 0)),
        pl.BlockSpec((D, 1), lambda i: (0, 0)),
    ],
    out_specs=pl.BlockSpec((D, B // _GR), lambda i: (0, i)),
)


def kernel(appearance_ids, table, W, b):
    ids = appearance_ids.astype(jnp.int32)
    emb_p = _sc_gather(ids, table.T)
    outT = _mm(emb_p, W, b.reshape(D, 1))
    return outT.T


# speedup vs baseline: 4.0408x; 1.1515x over previous
import functools

import jax
import jax.numpy as jnp
from jax import lax
from jax.experimental import pallas as pl
from jax.experimental.pallas import tpu as pltpu
from jax.experimental.pallas import tpu_sc as plsc

NUM_EMB = 1000000
D = 32
B = 16384
DP = 128

NC = 2
NS = 16
NW = NC * NS
BPW = B // NW
G = 8
NG = BPW // G
NSLOT = 3 * G

_mesh = plsc.VectorSubcoreMesh(core_axis_name="c", subcore_axis_name="s")


@functools.partial(
    pl.kernel,
    out_type=jax.ShapeDtypeStruct((B, DP), jnp.float32),
    mesh=_mesh,
    scratch_types=[
        pltpu.VMEM((BPW + 16,), jnp.int32),
        pltpu.VMEM((NSLOT, D, DP), jnp.float32),
        pltpu.VMEM((G, DP), jnp.float32),
        pltpu.SemaphoreType.DMA,
        pltpu.SemaphoreType.DMA,
        pltpu.SemaphoreType.DMA,
        pltpu.SemaphoreType.DMA,
    ],
    compiler_params=pltpu.CompilerParams(
        use_tc_tiling_on_sc=True, needs_layout_passes=False
    ),
)
def _sc_gather(ids_hbm, tabT_hbm, out_hbm, ids_v, sbuf, slab, semi, s0, s1, s2):
    wid = lax.axis_index("s") * NC + lax.axis_index("c")
    base = wid * BPW
    sems = [s0, s1, s2]
    pltpu.async_copy(
        ids_hbm.at[pl.ds(base, BPW)], ids_v.at[pl.ds(0, BPW)], semi
    ).wait()

    lanes = lax.iota(jnp.int32, 16)
    emask = lanes < G

    def fire(g, sem):
        idv = ids_v[pl.ds(g * G, 16)]
        for e in range(G):
            sid = idv[e]
            off = pl.multiple_of((sid >> 7) * 128, 128)
            slot = lax.rem(g * G + e, NSLOT)
            pltpu.async_copy(tabT_hbm.at[:, pl.ds(off, 128)], sbuf.at[slot], sem)

    def drain(g, sem):
        for e in range(G):
            slot = lax.rem(g * G + e, NSLOT)
            pltpu.make_async_copy(
                tabT_hbm.at[:, pl.ds(0, 128)], sbuf.at[slot], sem
            ).wait()

    def fire_sel(g):
        lax.switch(
            lax.rem(g, 3),
            [
                lambda: fire(g, s0),
                lambda: fire(g, s1),
                lambda: fire(g, s2),
            ],
        )

    def drain_sel(g):
        lax.switch(
            lax.rem(g, 3),
            [
                lambda: drain(g, s0),
                lambda: drain(g, s1),
                lambda: drain(g, s2),
            ],
        )

    fire(0, s0)
    fire(1, s1)

    def group(g, _):
        @pl.when(g + 2 < NG)
        def _():
            fire_sel(g + 2)

        drain_sel(g)
        idv = ids_v[pl.ds(g * G, 16)]
        r_vec = lax.bitwise_and(idv, jnp.int32(127))
        b_vec = lax.rem(g * G + lanes, jnp.int32(NSLOT))
        for j in range(D):
            j_vec = jnp.full((16,), j, jnp.int32)
            vals = plsc.load_gather(sbuf, [b_vec, j_vec, r_vec], mask=emask)
            plsc.store_scatter(slab, [lanes, j_vec], vals, mask=emask)
        pltpu.sync_copy(slab, out_hbm.at[pl.ds(base + g * G, G), :])
        return 0

    lax.fori_loop(0, NG, group, 0, unroll=False)


def _mm_body(x_ref, w_ref, b_ref, o_ref):
    xs = x_ref[...][:, 0:D]
    o_ref[...] = (
        lax.dot_general(
            w_ref[...], xs, (((1,), (1,)), ((), ())),
            preferred_element_type=jnp.float32,
        )
        + b_ref[...]
    )


_GR = 8
_mm = pl.pallas_call(
    _mm_body,
    out_shape=jax.ShapeDtypeStruct((D, B), jnp.float32),
    grid=(_GR,),
    in_specs=[
        pl.BlockSpec((B // _GR, DP), lambda i: (i, 0)),
        pl.BlockSpec((D, D), lambda i: (0, 0)),
        pl.BlockSpec((D, 1), lambda i: (0, 0)),
    ],
    out_specs=pl.BlockSpec((D, B // _GR), lambda i: (0, i)),
)


def kernel(appearance_ids, table, W, b):
    ids = appearance_ids.astype(jnp.int32)
    emb_p = _sc_gather(ids, table.T)
    outT = _mm(emb_p, W, b.reshape(D, 1))
    return outT.T
